# baseline (speedup 1.0000x reference)
import jax
import jax.numpy as jnp
from jax import lax
from jax.experimental import pallas as pl
from jax.experimental.pallas import tpu as pltpu
from jax.experimental.pallas import tpu_sc as plsc

B = 16384
L_SEQ = 200
VOCAB = 20
EMB = 4
NUM_OUT = 2

NUM_CORES = 2
NUM_SUBCORES = 16
LANES = 16
NUM_TILES = NUM_CORES * NUM_SUBCORES
ROWS_PER_TILE = B // NUM_TILES
CHUNK_ROWS = 128
CHUNKS = ROWS_PER_TILE // CHUNK_ROWS
GROUPS = CHUNK_ROWS // LANES
ROW_SHIFT = 7


def _tc_body(ids_hbm, tab_hbm, w_hbm, b_hbm, out_hbm, ids_v0, ids_v1,
             tab_v, w_v, b_v, counts, out_v, sem0, sem1):
  wid = lax.axis_index("s") * NUM_CORES + lax.axis_index("c")
  base = wid * ROWS_PER_TILE

  pltpu.sync_copy(tab_hbm, tab_v)
  pltpu.sync_copy(w_hbm, w_v)
  pltpu.sync_copy(b_hbm, b_v)

  wvec = w_v[:]
  w = [wvec[i] for i in range(NUM_OUT * EMB)]
  tvecs = [tab_v[pl.ds(j * LANES, LANES)] for j in range(VOCAB * EMB // LANES)]
  t_all = [tvecs[i // LANES][i % LANES] for i in range(VOCAB * EMB)]
  v0 = []
  v1 = []
  for k in range(VOCAB):
    t = t_all[k * EMB:(k + 1) * EMB]
    v0.append(t[0] * w[0] + t[1] * w[1] + t[2] * w[2] + t[3] * w[3])
    v1.append(t[0] * w[4] + t[1] * w[5] + t[2] * w[6] + t[3] * w[7])
  inv_l = 1.0 / L_SEQ
  bvec = b_v[:]
  b0 = bvec[0]
  b1 = bvec[1]

  lane = lax.iota(jnp.int32, LANES)
  ones_f = jnp.ones((LANES,), jnp.float32)
  zeros_f = jnp.zeros((LANES,), jnp.float32)
  row16s = [g * LANES + lane for g in range(GROUPS)]

  bufs = [ids_v0, ids_v1]
  sems = [sem0, sem1]
  handles = [None] * CHUNKS
  handles[0] = pltpu.async_copy(
      ids_hbm.at[pl.ds(base, CHUNK_ROWS)], bufs[0], sems[0])
  for ch in range(CHUNKS):
    if ch + 1 < CHUNKS:
      handles[ch + 1] = pltpu.async_copy(
          ids_hbm.at[pl.ds(base + (ch + 1) * CHUNK_ROWS, CHUNK_ROWS)],
          bufs[(ch + 1) % 2], sems[(ch + 1) % 2])
    handles[ch].wait()
    ids_v = bufs[ch % 2]

    @pl.loop(0, VOCAB * CHUNK_ROWS // LANES)
    def _zero(i):
      counts[pl.ds(i * LANES, LANES)] = zeros_f

    @plsc.parallel_loop(0, L_SEQ, unroll=8)
    def _tok(l):
      col = l + lane
      col = jnp.where(col >= L_SEQ, col - L_SEQ, col)
      for g in range(GROUPS):
        ids16 = plsc.load_gather(ids_v, [row16s[g], col])
        plsc.addupdate_scatter(counts, [(ids16 << ROW_SHIFT) | row16s[g]],
                               ones_f)

    @pl.loop(0, GROUPS)
    def _group(g):
      acc0 = jnp.zeros((LANES,), jnp.float32)
      acc1 = jnp.zeros((LANES,), jnp.float32)
      for k in range(VOCAB):
        cnt = counts[pl.ds(k * CHUNK_ROWS + g * LANES, LANES)]
        acc0 = acc0 + cnt * v0[k]
        acc1 = acc1 + cnt * v1[k]
      out_v[0, pl.ds(g * LANES, LANES)] = acc0 * inv_l + b0
      out_v[1, pl.ds(g * LANES, LANES)] = acc1 * inv_l + b1

    pltpu.sync_copy(out_v,
                    out_hbm.at[:, pl.ds(base + ch * CHUNK_ROWS, CHUNK_ROWS)])


@jax.jit
def _run(input_ids, table, W, b):
  ids = input_ids
  if ids.dtype != jnp.int32:
    ids = ids.astype(jnp.int32)
  tab_flat = table.reshape(-1).astype(jnp.float32)
  w_flat = jnp.pad(W.reshape(-1).astype(jnp.float32), (0, LANES - W.size))
  b_flat = jnp.pad(b.astype(jnp.float32), (0, LANES - b.size))
  mesh = plsc.VectorSubcoreMesh(
      core_axis_name="c", subcore_axis_name="s",
      num_cores=NUM_CORES, num_subcores=NUM_SUBCORES)
  fn = pl.kernel(
      _tc_body,
      out_type=jax.ShapeDtypeStruct((NUM_OUT, B), jnp.float32),
      mesh=mesh,
      compiler_params=pltpu.CompilerParams(
          use_tc_tiling_on_sc=True, needs_layout_passes=False),
      scratch_types=[
          pltpu.VMEM((CHUNK_ROWS, L_SEQ), jnp.int32),
          pltpu.VMEM((CHUNK_ROWS, L_SEQ), jnp.int32),
          pltpu.VMEM((VOCAB * EMB,), jnp.float32),
          pltpu.VMEM((LANES,), jnp.float32),
          pltpu.VMEM((LANES,), jnp.float32),
          pltpu.VMEM((VOCAB * CHUNK_ROWS,), jnp.float32),
          pltpu.VMEM((NUM_OUT, CHUNK_ROWS), jnp.float32),
          pltpu.SemaphoreType.DMA,
          pltpu.SemaphoreType.DMA,
      ],
  )
  return fn(ids, tab_flat, w_flat, b_flat).T


def kernel(input_ids, attention_mask, table, W, b):
  del attention_mask
  return _run(input_ids, table, W, b)


# speedup vs baseline: 1.0538x; 1.0538x over previous
import jax
import jax.numpy as jnp
from jax import lax
from jax.experimental import pallas as pl
from jax.experimental.pallas import tpu as pltpu
from jax.experimental.pallas import tpu_sc as plsc

B = 16384
L_SEQ = 200
VOCAB = 20
EMB = 4
NUM_OUT = 2

NUM_CORES = 2
NUM_SUBCORES = 16
LANES = 16
NUM_TILES = NUM_CORES * NUM_SUBCORES
ROWS_PER_TILE = B // NUM_TILES
CHUNK_ROWS = 128
CHUNKS = ROWS_PER_TILE // CHUNK_ROWS
GROUPS = CHUNK_ROWS // LANES
ROW_SHIFT = 7


def _tc_body(ids_hbm, tab_hbm, w_hbm, b_hbm, out_hbm, ids_v0, ids_v1,
             tab_v, w_v, b_v, counts, out_v, sem0, sem1):
  wid = lax.axis_index("s") * NUM_CORES + lax.axis_index("c")
  base = wid * ROWS_PER_TILE

  pltpu.sync_copy(tab_hbm, tab_v)
  pltpu.sync_copy(w_hbm, w_v)
  pltpu.sync_copy(b_hbm, b_v)

  wvec = w_v[:]
  w = [wvec[i] for i in range(NUM_OUT * EMB)]
  tvecs = [tab_v[pl.ds(j * LANES, LANES)] for j in range(VOCAB * EMB // LANES)]
  t_all = [tvecs[i // LANES][i % LANES] for i in range(VOCAB * EMB)]
  v0 = []
  v1 = []
  for k in range(VOCAB):
    t = t_all[k * EMB:(k + 1) * EMB]
    v0.append(t[0] * w[0] + t[1] * w[1] + t[2] * w[2] + t[3] * w[3])
    v1.append(t[0] * w[4] + t[1] * w[5] + t[2] * w[6] + t[3] * w[7])
  inv_l = 1.0 / L_SEQ
  bvec = b_v[:]
  b0 = bvec[0]
  b1 = bvec[1]

  lane = lax.iota(jnp.int32, LANES)
  ones_f = jnp.ones((LANES,), jnp.float32)
  zeros_f = jnp.zeros((LANES,), jnp.float32)
  row16s = [g * LANES + lane for g in range(GROUPS)]

  bufs = [ids_v0, ids_v1]
  sems = [sem0, sem1]
  handles = [None] * CHUNKS
  handles[0] = pltpu.async_copy(
      ids_hbm.at[pl.ds(base, CHUNK_ROWS)], bufs[0], sems[0])
  for ch in range(CHUNKS):
    if ch + 1 < CHUNKS:
      handles[ch + 1] = pltpu.async_copy(
          ids_hbm.at[pl.ds(base + (ch + 1) * CHUNK_ROWS, CHUNK_ROWS)],
          bufs[(ch + 1) % 2], sems[(ch + 1) % 2])
    handles[ch].wait()
    ids_v = bufs[ch % 2]

    @pl.loop(0, VOCAB * CHUNK_ROWS // LANES)
    def _zero(i):
      counts[pl.ds(i * LANES, LANES)] = zeros_f

    @plsc.parallel_loop(0, L_SEQ, unroll=2)
    def _tok(l):
      col = l + lane
      col = jnp.where(col >= L_SEQ, col - L_SEQ, col)
      for g in range(GROUPS):
        ids16 = plsc.load_gather(ids_v, [row16s[g], col])
        plsc.addupdate_scatter(counts, [(ids16 << ROW_SHIFT) | row16s[g]],
                               ones_f)

    @pl.loop(0, GROUPS)
    def _group(g):
      acc0 = jnp.zeros((LANES,), jnp.float32)
      acc1 = jnp.zeros((LANES,), jnp.float32)
      for k in range(VOCAB):
        cnt = counts[pl.ds(k * CHUNK_ROWS + g * LANES, LANES)]
        acc0 = acc0 + cnt * v0[k]
        acc1 = acc1 + cnt * v1[k]
      out_v[0, pl.ds(g * LANES, LANES)] = acc0 * inv_l + b0
      out_v[1, pl.ds(g * LANES, LANES)] = acc1 * inv_l + b1

    pltpu.sync_copy(out_v,
                    out_hbm.at[:, pl.ds(base + ch * CHUNK_ROWS, CHUNK_ROWS)])


@jax.jit
def _run(input_ids, table, W, b):
  ids = input_ids
  if ids.dtype != jnp.int32:
    ids = ids.astype(jnp.int32)
  tab_flat = table.reshape(-1).astype(jnp.float32)
  w_flat = jnp.pad(W.reshape(-1).astype(jnp.float32), (0, LANES - W.size))
  b_flat = jnp.pad(b.astype(jnp.float32), (0, LANES - b.size))
  mesh = plsc.VectorSubcoreMesh(
      core_axis_name="c", subcore_axis_name="s",
      num_cores=NUM_CORES, num_subcores=NUM_SUBCORES)
  fn = pl.kernel(
      _tc_body,
      out_type=jax.ShapeDtypeStruct((NUM_OUT, B), jnp.float32),
      mesh=mesh,
      compiler_params=pltpu.CompilerParams(
          use_tc_tiling_on_sc=True, needs_layout_passes=False),
      scratch_types=[
          pltpu.VMEM((CHUNK_ROWS, L_SEQ), jnp.int32),
          pltpu.VMEM((CHUNK_ROWS, L_SEQ), jnp.int32),
          pltpu.VMEM((VOCAB * EMB,), jnp.float32),
          pltpu.VMEM((LANES,), jnp.float32),
          pltpu.VMEM((LANES,), jnp.float32),
          pltpu.VMEM((VOCAB * CHUNK_ROWS,), jnp.float32),
          pltpu.VMEM((NUM_OUT, CHUNK_ROWS), jnp.float32),
          pltpu.SemaphoreType.DMA,
          pltpu.SemaphoreType.DMA,
      ],
  )
  return fn(ids, tab_flat, w_flat, b_flat).T


def kernel(input_ids, attention_mask, table, W, b):
  del attention_mask
  return _run(input_ids, table, W, b)


# speedup vs baseline: 1.0563x; 1.0024x over previous
import jax
import jax.numpy as jnp
from jax import lax
from jax.experimental import pallas as pl
from jax.experimental.pallas import tpu as pltpu
from jax.experimental.pallas import tpu_sc as plsc

B = 16384
L_SEQ = 200
VOCAB = 20
EMB = 4
NUM_OUT = 2

NUM_CORES = 2
NUM_SUBCORES = 16
LANES = 16
NUM_TILES = NUM_CORES * NUM_SUBCORES
ROWS_PER_TILE = B // NUM_TILES
CHUNK_ROWS = 128
CHUNKS = ROWS_PER_TILE // CHUNK_ROWS
GROUPS = CHUNK_ROWS // LANES
ROW_SHIFT = 7


def _tc_body(ids_hbm, tab_hbm, w_hbm, b_hbm, out_hbm, ids_v0, ids_v1,
             tab_v, w_v, b_v, counts, out_v, sem0, sem1):
  wid = lax.axis_index("s") * NUM_CORES + lax.axis_index("c")
  base = wid * ROWS_PER_TILE

  pltpu.sync_copy(tab_hbm, tab_v)
  pltpu.sync_copy(w_hbm, w_v)
  pltpu.sync_copy(b_hbm, b_v)

  wvec = w_v[:]
  w = [wvec[i] for i in range(NUM_OUT * EMB)]
  tvecs = [tab_v[pl.ds(j * LANES, LANES)] for j in range(VOCAB * EMB // LANES)]
  t_all = [tvecs[i // LANES][i % LANES] for i in range(VOCAB * EMB)]
  v0 = []
  v1 = []
  for k in range(VOCAB):
    t = t_all[k * EMB:(k + 1) * EMB]
    v0.append(t[0] * w[0] + t[1] * w[1] + t[2] * w[2] + t[3] * w[3])
    v1.append(t[0] * w[4] + t[1] * w[5] + t[2] * w[6] + t[3] * w[7])
  inv_l = 1.0 / L_SEQ
  bvec = b_v[:]
  b0 = bvec[0]
  b1 = bvec[1]

  lane = lax.iota(jnp.int32, LANES)
  ones_f = jnp.ones((LANES,), jnp.float32)
  zeros_f = jnp.zeros((LANES,), jnp.float32)
  row16s = [g * LANES + lane for g in range(GROUPS)]

  bufs = [ids_v0, ids_v1]
  sems = [sem0, sem1]
  handles = [None] * CHUNKS
  handles[0] = pltpu.async_copy(
      ids_hbm.at[pl.ds(base, CHUNK_ROWS)], bufs[0], sems[0])
  for ch in range(CHUNKS):
    if ch + 1 < CHUNKS:
      handles[ch + 1] = pltpu.async_copy(
          ids_hbm.at[pl.ds(base + (ch + 1) * CHUNK_ROWS, CHUNK_ROWS)],
          bufs[(ch + 1) % 2], sems[(ch + 1) % 2])
    handles[ch].wait()
    ids_v = bufs[ch % 2]

    @pl.loop(0, VOCAB * CHUNK_ROWS // LANES)
    def _zero(i):
      counts[pl.ds(i * LANES, LANES)] = zeros_f

    @plsc.parallel_loop(0, L_SEQ, unroll=1)
    def _tok(l):
      col = l + lane
      col = jnp.where(col >= L_SEQ, col - L_SEQ, col)
      for g in range(GROUPS):
        ids16 = plsc.load_gather(ids_v, [row16s[g], col])
        plsc.addupdate_scatter(counts, [(ids16 << ROW_SHIFT) | row16s[g]],
                               ones_f)

    @pl.loop(0, GROUPS)
    def _group(g):
      acc0 = jnp.zeros((LANES,), jnp.float32)
      acc1 = jnp.zeros((LANES,), jnp.float32)
      for k in range(VOCAB):
        cnt = counts[pl.ds(k * CHUNK_ROWS + g * LANES, LANES)]
        acc0 = acc0 + cnt * v0[k]
        acc1 = acc1 + cnt * v1[k]
      out_v[0, pl.ds(g * LANES, LANES)] = acc0 * inv_l + b0
      out_v[1, pl.ds(g * LANES, LANES)] = acc1 * inv_l + b1

    pltpu.sync_copy(out_v,
                    out_hbm.at[:, pl.ds(base + ch * CHUNK_ROWS, CHUNK_ROWS)])


@jax.jit
def _run(input_ids, table, W, b):
  ids = input_ids
  if ids.dtype != jnp.int32:
    ids = ids.astype(jnp.int32)
  tab_flat = table.reshape(-1).astype(jnp.float32)
  w_flat = jnp.pad(W.reshape(-1).astype(jnp.float32), (0, LANES - W.size))
  b_flat = jnp.pad(b.astype(jnp.float32), (0, LANES - b.size))
  mesh = plsc.VectorSubcoreMesh(
      core_axis_name="c", subcore_axis_name="s",
      num_cores=NUM_CORES, num_subcores=NUM_SUBCORES)
  fn = pl.kernel(
      _tc_body,
      out_type=jax.ShapeDtypeStruct((NUM_OUT, B), jnp.float32),
      mesh=mesh,
      compiler_params=pltpu.CompilerParams(
          use_tc_tiling_on_sc=True, needs_layout_passes=False),
      scratch_types=[
          pltpu.VMEM((CHUNK_ROWS, L_SEQ), jnp.int32),
          pltpu.VMEM((CHUNK_ROWS, L_SEQ), jnp.int32),
          pltpu.VMEM((VOCAB * EMB,), jnp.float32),
          pltpu.VMEM((LANES,), jnp.float32),
          pltpu.VMEM((LANES,), jnp.float32),
          pltpu.VMEM((VOCAB * CHUNK_ROWS,), jnp.float32),
          pltpu.VMEM((NUM_OUT, CHUNK_ROWS), jnp.float32),
          pltpu.SemaphoreType.DMA,
          pltpu.SemaphoreType.DMA,
      ],
  )
  return fn(ids, tab_flat, w_flat, b_flat).T


def kernel(input_ids, attention_mask, table, W, b):
  del attention_mask
  return _run(input_ids, table, W, b)
